# baseline (speedup 1.0000x reference)
import jax
import jax.numpy as jnp
from jax.experimental import pallas as pl


def _grid_rev_kernel(g_ref, out_ref):
    g = g_ref[0]

    def shift_lane(a, k):
        return jnp.concatenate([jnp.zeros((64, k), jnp.float32), a[:, :-k]], axis=1)

    def shift_x(a):
        return jnp.concatenate([jnp.zeros((1, 4096), jnp.float32), a[:-1]], axis=0)

    gz = g + shift_lane(g, 1)
    gy = g + shift_lane(g, 64)
    gzy = gz + shift_lane(gz, 64)

    wsum = gzy + shift_x(gzy)
    sx1 = gzy
    sy1 = gz + shift_x(gz)
    sz1 = gy + shift_x(gy)

    jxi = jax.lax.broadcasted_iota(jnp.int32, (64, 4096), 0)
    jq = jax.lax.broadcasted_iota(jnp.int32, (64, 4096), 1)
    jy = jq // 64
    jz = jq % 64

    interior = (jxi >= 1) & (jy >= 1) & (jz >= 1)
    mask = interior & (wsum > 0.0)
    r = 1.0 / jnp.where(mask, wsum, 1.0)
    scale = 1.0 / 32.0

    fx = jxi.astype(jnp.float32)
    fy = jy.astype(jnp.float32)
    fz = jz.astype(jnp.float32)

    px = jnp.where(mask, ((fx - 33.0) + sx1 * r) * scale, 0.0)
    py = jnp.where(mask, ((fy - 33.0) + sy1 * r) * scale, 0.0)
    pz = jnp.where(mask, ((fz - 33.0) + sz1 * r) * scale, 0.0)

    lane = jax.lax.broadcasted_iota(jnp.int32, (64, 128), 1)
    idx = []
    selects = []
    for v in range(3):
        j = 128 * v + lane
        idx.append(j // 3)
        cm = j % 3
        selects.append((cm == 0, cm == 1))
    for t in range(32):
        lo, hi = 128 * t, 128 * (t + 1)
        sx_, sy_, sz_ = px[:, lo:hi], py[:, lo:hi], pz[:, lo:hi]
        for v in range(3):
            g0 = jnp.take_along_axis(sx_, idx[v], axis=1)
            g1 = jnp.take_along_axis(sy_, idx[v], axis=1)
            g2 = jnp.take_along_axis(sz_, idx[v], axis=1)
            m0, m1 = selects[v]
            out = jnp.where(m0, g0, jnp.where(m1, g1, g2))
            base = 384 * t + 128 * v
            out_ref[0, :, base:base + 128] = out


def kernel(grid):
    B = grid.shape[0]
    g2 = grid.reshape(B, 64, 4096)
    out = pl.pallas_call(
        _grid_rev_kernel,
        grid=(B,),
        in_specs=[pl.BlockSpec((1, 64, 4096), lambda b: (b, 0, 0))],
        out_specs=pl.BlockSpec((1, 64, 12288), lambda b: (b, 0, 0)),
        out_shape=jax.ShapeDtypeStruct((B, 64, 12288), jnp.float32),
    )(g2)
    return out.reshape(B, 64 * 64 * 64, 3)


# speedup vs baseline: 19.4958x; 19.4958x over previous
import jax
import jax.numpy as jnp
from jax.experimental import pallas as pl

_SLAB = 8192
_HALO = 4096


def _grid_rev_kernel(slab_ref, halo_ref, out_ref):
    i = pl.program_id(0)
    b = slab_ref.shape[0]
    halo = halo_ref[...].reshape(b, _HALO)
    slab = slab_ref[...].reshape(b, _SLAB)
    w = jnp.concatenate([halo, slab], axis=1)

    def sh(a, k):
        return jnp.concatenate([jnp.zeros((b, k), jnp.float32), a[:, :-k]], axis=1)

    gz = w + sh(w, 1)
    gy = w + sh(w, 64)
    gzy = gz + sh(gz, 64)

    wsum = (gzy + sh(gzy, 4096))[:, _HALO:]
    sx1 = gzy[:, _HALO:]
    sy1 = (gz + sh(gz, 4096))[:, _HALO:]
    sz1 = (gy + sh(gy, 4096))[:, _HALO:]

    l = jax.lax.broadcasted_iota(jnp.int32, (b, _SLAB), 1)
    jz = l % 64
    jy = (l // 64) % 64
    jx = (l // 4096) + 2 * i

    interior = (jx >= 1) & (jy >= 1) & (jz >= 1)
    mask = interior & (wsum > 0.0)
    r = 1.0 / jnp.where(mask, wsum, 1.0)
    scale = 1.0 / 32.0

    fx = jx.astype(jnp.float32)
    fy = jy.astype(jnp.float32)
    fz = jz.astype(jnp.float32)

    out_ref[0] = jnp.where(mask, ((fx - 33.0) + sx1 * r) * scale, 0.0)
    out_ref[1] = jnp.where(mask, ((fy - 33.0) + sy1 * r) * scale, 0.0)
    out_ref[2] = jnp.where(mask, ((fz - 33.0) + sz1 * r) * scale, 0.0)


def kernel(grid):
    B = grid.shape[0]
    gv = grid.reshape(B, 4096, 64)
    out = pl.pallas_call(
        _grid_rev_kernel,
        grid=(64 * 4096 // _SLAB,),
        in_specs=[
            pl.BlockSpec((B, _SLAB // 64, 64), lambda i: (0, i, 0)),
            pl.BlockSpec((B, _HALO // 64, 64),
                         lambda i: (0, jnp.maximum(2 * i - 1, 0), 0)),
        ],
        out_specs=pl.BlockSpec((3, B, _SLAB), lambda i: (0, 0, i)),
        out_shape=jax.ShapeDtypeStruct((3, B, 64 * 64 * 64), jnp.float32),
    )(gv, gv)
    return out.transpose(1, 2, 0)


# speedup vs baseline: 24.6968x; 1.2668x over previous
import jax
import jax.numpy as jnp
from jax.experimental import pallas as pl

_SLAB = 16384
_HALO = 4096


def _grid_rev_kernel(slab_ref, halo_ref, out_ref):
    i = pl.program_id(0)
    b = slab_ref.shape[0]
    halo = halo_ref[...].reshape(b, _HALO)
    slab = slab_ref[...].reshape(b, _SLAB)
    w = jnp.concatenate([halo, slab], axis=1)

    def sh(a, k):
        return jnp.concatenate([jnp.zeros((b, k), jnp.float32), a[:, :-k]], axis=1)

    gz = w + sh(w, 1)
    gy = w + sh(w, 64)
    gzy = gz + sh(gz, 64)

    wsum = gzy[:, _HALO:] + gzy[:, :_SLAB]
    sx1 = gzy[:, _HALO:]
    sy1 = gz[:, _HALO:] + gz[:, :_SLAB]
    sz1 = gy[:, _HALO:] + gy[:, :_SLAB]

    l = jax.lax.broadcasted_iota(jnp.int32, (b, _SLAB), 1)
    jz = l % 64
    jy = (l // 64) % 64
    jx = (l // 4096) + (_SLAB // 4096) * i

    interior = (jx >= 1) & (jy >= 1) & (jz >= 1)
    mask = interior & (wsum > 0.0)
    rs = (1.0 / 32.0) / jnp.where(mask, wsum, 1.0)
    fxs = (jx.astype(jnp.float32) - 33.0) * (1.0 / 32.0)
    fys = (jy.astype(jnp.float32) - 33.0) * (1.0 / 32.0)
    fzs = (jz.astype(jnp.float32) - 33.0) * (1.0 / 32.0)

    out_ref[0] = jnp.where(mask, fxs + sx1 * rs, 0.0)
    out_ref[1] = jnp.where(mask, fys + sy1 * rs, 0.0)
    out_ref[2] = jnp.where(mask, fzs + sz1 * rs, 0.0)


def kernel(grid):
    B = grid.shape[0]
    gv = grid.reshape(B, 4096, 64)
    out = pl.pallas_call(
        _grid_rev_kernel,
        grid=(64 * 4096 // _SLAB,),
        in_specs=[
            pl.BlockSpec((B, _SLAB // 64, 64), lambda i: (0, i, 0)),
            pl.BlockSpec((B, _HALO // 64, 64),
                         lambda i: (0, jnp.maximum((_SLAB // 4096) * i - 1, 0), 0)),
        ],
        out_specs=pl.BlockSpec((3, B, _SLAB), lambda i: (0, 0, i)),
        out_shape=jax.ShapeDtypeStruct((3, B, 64 * 64 * 64), jnp.float32),
    )(gv, gv)
    return out.transpose(1, 2, 0)


# speedup vs baseline: 27.5281x; 1.1146x over previous
import jax
import jax.numpy as jnp
from jax.experimental import pallas as pl

_SLAB = 32768
_HALO = 4096


def _grid_rev_kernel(slab_ref, halo_ref, out_ref):
    i = pl.program_id(0)
    b = slab_ref.shape[0]
    halo = halo_ref[...].reshape(b, _HALO)
    slab = slab_ref[...].reshape(b, _SLAB)
    w = jnp.concatenate([halo, slab], axis=1)

    def sh(a, k):
        return jnp.concatenate([jnp.zeros((b, k), jnp.float32), a[:, :-k]], axis=1)

    gz = w + sh(w, 1)
    gy = w + sh(w, 64)
    gzy = gz + sh(gz, 64)

    wsum = gzy[:, _HALO:] + gzy[:, :_SLAB]
    sx1 = gzy[:, _HALO:]
    sy1 = gz[:, _HALO:] + gz[:, :_SLAB]
    sz1 = gy[:, _HALO:] + gy[:, :_SLAB]

    l = jax.lax.broadcasted_iota(jnp.int32, (b, _SLAB), 1)
    jz = l % 64
    jy = (l // 64) % 64
    jx = (l // 4096) + (_SLAB // 4096) * i

    interior = (jx >= 1) & (jy >= 1) & (jz >= 1)
    mask = interior & (wsum > 0.0)
    rs = (1.0 / 32.0) / jnp.where(mask, wsum, 1.0)
    fxs = (jx.astype(jnp.float32) - 33.0) * (1.0 / 32.0)
    fys = (jy.astype(jnp.float32) - 33.0) * (1.0 / 32.0)
    fzs = (jz.astype(jnp.float32) - 33.0) * (1.0 / 32.0)

    out_ref[0] = jnp.where(mask, fxs + sx1 * rs, 0.0)
    out_ref[1] = jnp.where(mask, fys + sy1 * rs, 0.0)
    out_ref[2] = jnp.where(mask, fzs + sz1 * rs, 0.0)


def kernel(grid):
    B = grid.shape[0]
    gv = grid.reshape(B, 4096, 64)
    out = pl.pallas_call(
        _grid_rev_kernel,
        grid=(64 * 4096 // _SLAB,),
        in_specs=[
            pl.BlockSpec((B, _SLAB // 64, 64), lambda i: (0, i, 0)),
            pl.BlockSpec((B, _HALO // 64, 64),
                         lambda i: (0, jnp.maximum((_SLAB // 4096) * i - 1, 0), 0)),
        ],
        out_specs=pl.BlockSpec((3, B, _SLAB), lambda i: (0, 0, i)),
        out_shape=jax.ShapeDtypeStruct((3, B, 64 * 64 * 64), jnp.float32),
    )(gv, gv)
    return out.transpose(1, 2, 0)
